# baseline (speedup 1.0000x reference)
import jax
import jax.numpy as jnp
from jax import lax
from jax.experimental import pallas as pl
from jax.experimental.pallas import tpu as pltpu
from jax.experimental.pallas import tpu_sc as plsc

VOCAB = 100000
EMBED = 32
BATCH = 1024

_NC = 2
_NS = 16
_NW = _NC * _NS

_CHUNK = 128
_NCHUNK = BATCH // _CHUNK
_LANES = 16

_V_TILE = 4096


def _gather_body(flat_hbm, idx_hbm, outT_hbm, out_v, sem, idx_v):
    wid = lax.axis_index("s") * _NC + lax.axis_index("c")
    off = wid * VOCAB
    pltpu.sync_copy(idx_hbm, idx_v)
    copies = []
    for j in range(_NCHUNK):
        for i in range(_CHUNK // _LANES):
            sl = pl.ds(j * _CHUNK + i * _LANES, _LANES)
            idx_v[sl] = idx_v[sl] + off
        copies.append(
            pltpu.async_copy(
                flat_hbm.at[idx_v.at[pl.ds(j * _CHUNK, _CHUNK)]],
                out_v.at[pl.ds(j * _CHUNK, _CHUNK)],
                sem,
            )
        )
    for c in copies:
        c.wait()
    pltpu.sync_copy(out_v, outT_hbm.at[wid])


@jax.jit
def _sc_gather(flat_table, idx):
    mesh = plsc.VectorSubcoreMesh(core_axis_name="c", subcore_axis_name="s")
    return pl.kernel(
        _gather_body,
        mesh=mesh,
        out_type=jax.ShapeDtypeStruct((EMBED, BATCH), jnp.float32),
        scratch_types=[
            pltpu.VMEM((BATCH,), jnp.float32),
            pltpu.SemaphoreType.DMA,
            pltpu.VMEM((BATCH,), jnp.int32),
        ],
        compiler_params=pltpu.CompilerParams(use_tc_tiling_on_sc=False),
    )(flat_table, idx)


def _proj_body(wt_ref, yt_ref, b_ref, o_ref):
    ones = jnp.ones((1, BATCH), dtype=jnp.float32)
    o_ref[...] = (
        lax.dot_general(
            wt_ref[...],
            yt_ref[...],
            (((0,), (0,)), ((), ())),
            preferred_element_type=jnp.float32,
        )
        + lax.dot_general(
            b_ref[...],
            ones,
            (((0,), (0,)), ((), ())),
            preferred_element_type=jnp.float32,
        )
    )


@jax.jit
def _tc_project(wt, yt, b2):
    grid = pl.cdiv(VOCAB, _V_TILE)
    return pl.pallas_call(
        _proj_body,
        grid=(grid,),
        in_specs=[
            pl.BlockSpec((EMBED, _V_TILE), lambda j: (0, j)),
            pl.BlockSpec((EMBED, BATCH), lambda j: (0, 0)),
            pl.BlockSpec((1, _V_TILE), lambda j: (0, j)),
        ],
        out_specs=pl.BlockSpec((_V_TILE, BATCH), lambda j: (j, 0)),
        out_shape=jax.ShapeDtypeStruct((VOCAB, BATCH), jnp.float32),
    )(wt, yt, b2)


def kernel(batch, embed_table, W, b):
    flat = embed_table.T.reshape(-1)
    yt = _sc_gather(flat, batch.astype(jnp.int32))
    outT = _tc_project(W.T, yt, b.reshape(1, VOCAB))
    return outT.T


# speedup vs baseline: 1.0035x; 1.0035x over previous
import jax
import jax.numpy as jnp
from jax import lax
from jax.experimental import pallas as pl
from jax.experimental.pallas import tpu as pltpu
from jax.experimental.pallas import tpu_sc as plsc

VOCAB = 100000
EMBED = 32
BATCH = 1024

_NC = 2
_NS = 16
_NW = _NC * _NS

_CHUNK = 128
_NCHUNK = BATCH // _CHUNK
_LANES = 16

_V_TILE = 4096

_LTV = 32768
_LQ = _LTV // 128
_LG2 = (VOCAB + _LTV - 1) // _LTV
_LBLK = 8 * _LTV
_LR = EMBED // 8
_FLAT = _LR * _LG2 * _LBLK
_LSHIFT = _LTV.bit_length() - 1


def _lin_body(t_ref, o_ref):
    x = t_ref[...]
    for q in range(_LQ):
        o_ref[pl.ds(q * 1024, 1024)] = x[:, q * 128 : (q + 1) * 128].reshape(1024)


@jax.jit
def _tc_linearize(tableT):
    return pl.pallas_call(
        _lin_body,
        grid=(_LR, _LG2),
        in_specs=[pl.BlockSpec((8, _LTV), lambda r, j: (r, j))],
        out_specs=pl.BlockSpec((_LBLK,), lambda r, j: (r * _LG2 + j,)),
        out_shape=jax.ShapeDtypeStruct((_FLAT,), jnp.float32),
    )(tableT)


def _gather_body(flat_hbm, idx_hbm, outT_hbm, out_v, sem, idx_v):
    wid = lax.axis_index("s") * _NC + lax.axis_index("c")
    base = (wid // 8) * (_LG2 * _LBLK) + (wid % 8) * 128
    pltpu.sync_copy(idx_hbm, idx_v)
    copies = []
    for j in range(_NCHUNK):
        for i in range(_CHUNK // _LANES):
            sl = pl.ds(j * _CHUNK + i * _LANES, _LANES)
            v = idx_v[sl]
            idx_v[sl] = (
                (v >> _LSHIFT) * _LBLK
                + ((v >> 7) & (_LQ - 1)) * 1024
                + (v & 127)
                + base
            )
        copies.append(
            pltpu.async_copy(
                flat_hbm.at[idx_v.at[pl.ds(j * _CHUNK, _CHUNK)]],
                out_v.at[pl.ds(j * _CHUNK, _CHUNK)],
                sem,
            )
        )
    for c in copies:
        c.wait()
    pltpu.sync_copy(out_v, outT_hbm.at[wid])


@jax.jit
def _sc_gather(flat_table, idx):
    mesh = plsc.VectorSubcoreMesh(core_axis_name="c", subcore_axis_name="s")
    return pl.kernel(
        _gather_body,
        mesh=mesh,
        out_type=jax.ShapeDtypeStruct((EMBED, BATCH), jnp.float32),
        scratch_types=[
            pltpu.VMEM((BATCH,), jnp.float32),
            pltpu.SemaphoreType.DMA,
            pltpu.VMEM((BATCH,), jnp.int32),
        ],
        compiler_params=pltpu.CompilerParams(use_tc_tiling_on_sc=False),
    )(flat_table, idx)


def _proj_body(wt_ref, yt_ref, b_ref, o_ref):
    ones = jnp.ones((1, BATCH), dtype=jnp.float32)
    o_ref[...] = (
        lax.dot_general(
            wt_ref[...],
            yt_ref[...],
            (((0,), (0,)), ((), ())),
            preferred_element_type=jnp.float32,
        )
        + lax.dot_general(
            b_ref[...],
            ones,
            (((0,), (0,)), ((), ())),
            preferred_element_type=jnp.float32,
        )
    )


@jax.jit
def _tc_project(wt, yt, b2):
    grid = pl.cdiv(VOCAB, _V_TILE)
    return pl.pallas_call(
        _proj_body,
        grid=(grid,),
        in_specs=[
            pl.BlockSpec((EMBED, _V_TILE), lambda j: (0, j)),
            pl.BlockSpec((EMBED, BATCH), lambda j: (0, 0)),
            pl.BlockSpec((1, _V_TILE), lambda j: (0, j)),
        ],
        out_specs=pl.BlockSpec((_V_TILE, BATCH), lambda j: (j, 0)),
        out_shape=jax.ShapeDtypeStruct((VOCAB, BATCH), jnp.float32),
    )(wt, yt, b2)


def kernel(batch, embed_table, W, b):
    flat = _tc_linearize(embed_table.T)
    yt = _sc_gather(flat, batch.astype(jnp.int32))
    outT = _tc_project(W.T, yt, b.reshape(1, VOCAB))
    return outT.T


# speedup vs baseline: 1.0403x; 1.0367x over previous
import jax
import jax.numpy as jnp
from jax import lax
from jax.experimental import pallas as pl
from jax.experimental.pallas import tpu as pltpu
from jax.experimental.pallas import tpu_sc as plsc

VOCAB = 100000
EMBED = 32
BATCH = 1024

_NC = 2
_NS = 16
_NW = _NC * _NS

_CHUNK = 128
_NCHUNK = BATCH // _CHUNK
_LANES = 16

_V_TILE = 4096

_LTV = 51200
_LQ = _LTV // 128
_LG2 = (VOCAB + _LTV - 1) // _LTV
_LBLK = 8 * _LTV
_LR = EMBED // 8
_FLAT = _LR * _LG2 * _LBLK


def _lin_body(t_ref, o_ref):
    x = t_ref[...]
    for q in range(_LQ):
        o_ref[pl.ds(q * 1024, 1024)] = x[:, q * 128 : (q + 1) * 128].reshape(1024)


@jax.jit
def _tc_linearize(tableT):
    return pl.pallas_call(
        _lin_body,
        grid=(_LR, _LG2),
        in_specs=[pl.BlockSpec((8, _LTV), lambda r, j: (r, j))],
        out_specs=pl.BlockSpec((_LBLK,), lambda r, j: (r * _LG2 + j,)),
        out_shape=jax.ShapeDtypeStruct((_FLAT,), jnp.float32),
    )(tableT)


def _gather_body(flat_hbm, idx_hbm, outT_hbm, out_v, sem, idx_v):
    wid = lax.axis_index("s") * _NC + lax.axis_index("c")
    base = (wid // 8) * (_LG2 * _LBLK) + (wid % 8) * 128
    pltpu.sync_copy(idx_hbm, idx_v)
    copies = []
    for j in range(_NCHUNK):
        for i in range(_CHUNK // _LANES):
            sl = pl.ds(j * _CHUNK + i * _LANES, _LANES)
            v = idx_v[sl]
            pg = v >> 7
            idx_v[sl] = (
                jnp.where(pg >= _LQ, _LBLK + (pg - _LQ) * 1024, pg * 1024)
                + (v & 127)
                + base
            )
        copies.append(
            pltpu.async_copy(
                flat_hbm.at[idx_v.at[pl.ds(j * _CHUNK, _CHUNK)]],
                out_v.at[pl.ds(j * _CHUNK, _CHUNK)],
                sem,
            )
        )
    for c in copies:
        c.wait()
    pltpu.sync_copy(out_v, outT_hbm.at[wid])


@jax.jit
def _sc_gather(flat_table, idx):
    mesh = plsc.VectorSubcoreMesh(core_axis_name="c", subcore_axis_name="s")
    return pl.kernel(
        _gather_body,
        mesh=mesh,
        out_type=jax.ShapeDtypeStruct((EMBED, BATCH), jnp.float32),
        scratch_types=[
            pltpu.VMEM((BATCH,), jnp.float32),
            pltpu.SemaphoreType.DMA,
            pltpu.VMEM((BATCH,), jnp.int32),
        ],
        compiler_params=pltpu.CompilerParams(use_tc_tiling_on_sc=False),
    )(flat_table, idx)


def _proj_body(wt_ref, yt_ref, b_ref, o_ref):
    ones = jnp.ones((1, BATCH), dtype=jnp.float32)
    o_ref[...] = (
        lax.dot_general(
            wt_ref[...],
            yt_ref[...],
            (((0,), (0,)), ((), ())),
            preferred_element_type=jnp.float32,
        )
        + lax.dot_general(
            b_ref[...],
            ones,
            (((0,), (0,)), ((), ())),
            preferred_element_type=jnp.float32,
        )
    )


@jax.jit
def _tc_project(wt, yt, b2):
    grid = pl.cdiv(VOCAB, _V_TILE)
    return pl.pallas_call(
        _proj_body,
        grid=(grid,),
        in_specs=[
            pl.BlockSpec((EMBED, _V_TILE), lambda j: (0, j)),
            pl.BlockSpec((EMBED, BATCH), lambda j: (0, 0)),
            pl.BlockSpec((1, _V_TILE), lambda j: (0, j)),
        ],
        out_specs=pl.BlockSpec((_V_TILE, BATCH), lambda j: (j, 0)),
        out_shape=jax.ShapeDtypeStruct((VOCAB, BATCH), jnp.float32),
    )(wt, yt, b2)


def kernel(batch, embed_table, W, b):
    flat = _tc_linearize(embed_table.T)
    yt = _sc_gather(flat, batch.astype(jnp.int32))
    outT = _tc_project(W.T, yt, b.reshape(1, VOCAB))
    return outT.T


# speedup vs baseline: 1.0476x; 1.0070x over previous
import jax
import jax.numpy as jnp
from jax import lax
from jax.experimental import pallas as pl
from jax.experimental.pallas import tpu as pltpu
from jax.experimental.pallas import tpu_sc as plsc

VOCAB = 100000
EMBED = 32
BATCH = 1024

_NC = 2
_NS = 16
_NW = _NC * _NS

_CHUNK = 128
_NCHUNK = BATCH // _CHUNK
_LANES = 16

_V_TILE = 4096

_LTV = 51200
_LQ = _LTV // 128
_LG2 = (VOCAB + _LTV - 1) // _LTV
_LROWS = 16
_LBLK = _LROWS * _LTV
_LHALF = 8 * _LTV
_LR = EMBED // _LROWS
_FLAT = _LR * _LG2 * _LBLK


def _lin_body(t_ref, o_ref):
    x = t_ref[...]
    for h in range(_LROWS // 8):
        for q in range(_LQ):
            o_ref[pl.ds((h * _LQ + q) * 1024, 1024)] = x[
                h * 8 : (h + 1) * 8, q * 128 : (q + 1) * 128
            ].reshape(1024)


@jax.jit
def _tc_linearize(tableT):
    return pl.pallas_call(
        _lin_body,
        grid=(_LR, _LG2),
        in_specs=[pl.BlockSpec((_LROWS, _LTV), lambda r, j: (r, j))],
        out_specs=pl.BlockSpec((_LBLK,), lambda r, j: (r * _LG2 + j,)),
        out_shape=jax.ShapeDtypeStruct((_FLAT,), jnp.float32),
    )(tableT)


def _gather_body(flat_hbm, idx_hbm, outT_hbm, out_v, sem, idx_v):
    wid = lax.axis_index("s") * _NC + lax.axis_index("c")
    base = (
        (wid // _LROWS) * (_LG2 * _LBLK)
        + ((wid % _LROWS) // 8) * _LHALF
        + (wid % 8) * 128
    )
    pltpu.sync_copy(idx_hbm, idx_v)
    copies = []
    for j in range(_NCHUNK):
        for i in range(_CHUNK // _LANES):
            sl = pl.ds(j * _CHUNK + i * _LANES, _LANES)
            v = idx_v[sl]
            pg = v >> 7
            idx_v[sl] = (
                jnp.where(pg >= _LQ, _LBLK + (pg - _LQ) * 1024, pg * 1024)
                + (v & 127)
                + base
            )
        copies.append(
            pltpu.async_copy(
                flat_hbm.at[idx_v.at[pl.ds(j * _CHUNK, _CHUNK)]],
                out_v.at[pl.ds(j * _CHUNK, _CHUNK)],
                sem,
            )
        )
    for c in copies:
        c.wait()
    pltpu.sync_copy(out_v, outT_hbm.at[wid])


@jax.jit
def _sc_gather(flat_table, idx):
    mesh = plsc.VectorSubcoreMesh(core_axis_name="c", subcore_axis_name="s")
    return pl.kernel(
        _gather_body,
        mesh=mesh,
        out_type=jax.ShapeDtypeStruct((EMBED, BATCH), jnp.float32),
        scratch_types=[
            pltpu.VMEM((BATCH,), jnp.float32),
            pltpu.SemaphoreType.DMA,
            pltpu.VMEM((BATCH,), jnp.int32),
        ],
        compiler_params=pltpu.CompilerParams(use_tc_tiling_on_sc=False),
    )(flat_table, idx)


def _proj_body(wt_ref, yt_ref, b_ref, o_ref):
    ones = jnp.ones((1, BATCH), dtype=jnp.float32)
    o_ref[...] = (
        lax.dot_general(
            wt_ref[...],
            yt_ref[...],
            (((0,), (0,)), ((), ())),
            preferred_element_type=jnp.float32,
        )
        + lax.dot_general(
            b_ref[...],
            ones,
            (((0,), (0,)), ((), ())),
            preferred_element_type=jnp.float32,
        )
    )


@jax.jit
def _tc_project(wt, yt, b2):
    grid = pl.cdiv(VOCAB, _V_TILE)
    return pl.pallas_call(
        _proj_body,
        grid=(grid,),
        in_specs=[
            pl.BlockSpec((EMBED, _V_TILE), lambda j: (0, j)),
            pl.BlockSpec((EMBED, BATCH), lambda j: (0, 0)),
            pl.BlockSpec((1, _V_TILE), lambda j: (0, j)),
        ],
        out_specs=pl.BlockSpec((_V_TILE, BATCH), lambda j: (j, 0)),
        out_shape=jax.ShapeDtypeStruct((VOCAB, BATCH), jnp.float32),
    )(wt, yt, b2)


def kernel(batch, embed_table, W, b):
    flat = _tc_linearize(embed_table.T)
    yt = _sc_gather(flat, batch.astype(jnp.int32))
    outT = _tc_project(W.T, yt, b.reshape(1, VOCAB))
    return outT.T
